# Initial kernel scaffold; baseline (speedup 1.0000x reference)
import jax
import jax.numpy as jnp
from jax.experimental import pallas as pl


def kernel(x, edge_index, edges_weight, W_gcn, b_gcn, W1, b1, W2, b2, W3, b3):
    raise NotImplementedError("write your pallas kernel here")


# speedup vs baseline: 5.0306x; 5.0306x over previous
import functools

import jax
import jax.numpy as jnp
from jax import lax
from jax.experimental import pallas as pl
from jax.experimental.pallas import tpu as pltpu
from jax.experimental.pallas import tpu_sc as plsc

N = 10000
NPAD = 10240
E = 320000
F_IN = 128
H1, H2, H3, C = 64, 32, 16, 4

HP = 128

NC, NS, L = 2, 16, 16
NW = NC * NS
B = 128
WIN = 80
EPT = B * WIN
E_PAD = EPT * NW
SEG = NPAD // NS

_mesh = plsc.VectorSubcoreMesh(core_axis_name="c", subcore_axis_name="s")


@functools.partial(
    pl.kernel,
    out_type=jax.ShapeDtypeStruct((E_PAD, HP), jnp.float32),
    mesh=_mesh,
    scratch_types=[
        pltpu.VMEM((B,), jnp.int32),
        pltpu.VMEM((B, HP), jnp.float32),
        pltpu.SemaphoreType.DMA,
    ],
)
def _gather_kernel(ht_h, src_h, out, idx_v, rows_v, sem):
  c = lax.axis_index("c")
  s = lax.axis_index("s")
  wid = s * NC + c
  base = wid * EPT

  def body(w, carry):
    off = base + w * B
    pltpu.sync_copy(src_h.at[pl.ds(off, B)], idx_v)
    pltpu.async_copy(ht_h.at[idx_v], rows_v, sem).wait()
    pltpu.sync_copy(rows_v, out.at[pl.ds(off, B)])
    return carry

  lax.fori_loop(0, WIN, body, 0)


@functools.partial(
    pl.kernel,
    out_type=jax.ShapeDtypeStruct((NC * NPAD, HP), jnp.float32),
    mesh=_mesh,
    scratch_types=[
        pltpu.VMEM((B,), jnp.int32),
        pltpu.VMEM((B, HP), jnp.float32),
        pltpu.VMEM_SHARED((NPAD, HP), jnp.float32),
    ],
)
def _scatter_kernel(msg_h, dst_h, zero_h, out, idx_v, val_v, agg_sh):
  c = lax.axis_index("c")
  s = lax.axis_index("s")
  wid = s * NC + c
  base = wid * EPT

  pltpu.sync_copy(zero_h.at[pl.ds(0, B)], val_v)
  for k in range(SEG // B):
    pltpu.sync_copy(val_v, agg_sh.at[pl.ds(s * SEG + k * B, B)])
  plsc.subcore_barrier()

  def body(w, carry):
    off = base + w * B
    pltpu.sync_copy(dst_h.at[pl.ds(off, B)], idx_v)
    pltpu.sync_copy(msg_h.at[pl.ds(off, B)], val_v)
    pltpu.sync_copy(val_v, agg_sh.at[idx_v], add=True)
    return carry

  lax.fori_loop(0, WIN, body, 0)
  plsc.subcore_barrier()
  for k in range(SEG // B):
    pltpu.sync_copy(agg_sh.at[pl.ds(s * SEG + k * B, B)], val_v)
    pltpu.sync_copy(val_v, out.at[pl.ds(c * NPAD + s * SEG + k * B, B)])


def _pre_body(x_ref, w_ref, dega_ref, degb_ref, ht_ref, htp_ref, dinv_ref):
  deg = dega_ref[...] + degb_ref[...] + 1.0
  dinv = lax.rsqrt(deg)
  h = jnp.dot(x_ref[...], w_ref[...], preferred_element_type=jnp.float32)
  hd = h * dinv
  ht_ref[...] = hd
  htp_ref[...] = jnp.concatenate([hd, jnp.zeros_like(hd)], axis=1)
  dinv_ref[...] = dinv


_pre_call = pl.pallas_call(
    _pre_body,
    out_shape=[
        jax.ShapeDtypeStruct((N, H1), jnp.float32),
        jax.ShapeDtypeStruct((N, HP), jnp.float32),
        jax.ShapeDtypeStruct((N, 1), jnp.float32),
    ],
)

_SCALE_ROWS = 4096


def _scale_body(msg_ref, ew_ref, out_ref):
  out_ref[...] = msg_ref[...] * ew_ref[...]


_scale_call = pl.pallas_call(
    _scale_body,
    grid=(E_PAD // _SCALE_ROWS,),
    in_specs=[
        pl.BlockSpec((_SCALE_ROWS, HP), lambda i: (i, 0)),
        pl.BlockSpec((_SCALE_ROWS, 1), lambda i: (i, 0)),
    ],
    out_specs=pl.BlockSpec((_SCALE_ROWS, HP), lambda i: (i, 0)),
    out_shape=jax.ShapeDtypeStruct((E_PAD, HP), jnp.float32),
)


def _post_body(agga_ref, aggb_ref, ht_ref, dinv_ref, bg_ref,
               w1_ref, b1_ref, w2_ref, b2_ref, w3_ref, b3_ref, out_ref):
  a = (agga_ref[:, :H1] + aggb_ref[:, :H1] + ht_ref[...]) * dinv_ref[...]
  a = jnp.maximum(a + bg_ref[...], 0.0)
  a = jnp.maximum(
      jnp.dot(a, w1_ref[...], preferred_element_type=jnp.float32)
      + b1_ref[...], 0.0)
  a = jnp.maximum(
      jnp.dot(a, w2_ref[...], preferred_element_type=jnp.float32)
      + b2_ref[...], 0.0)
  z = jnp.dot(a, w3_ref[...], preferred_element_type=jnp.float32) + b3_ref[...]
  m = jnp.max(z, axis=1, keepdims=True)
  lse = jnp.log(jnp.sum(jnp.exp(z - m), axis=1, keepdims=True)) + m
  out_ref[...] = z - lse


_post_call = pl.pallas_call(
    _post_body,
    out_shape=jax.ShapeDtypeStruct((N, C), jnp.float32),
)


def kernel(x, edge_index, edges_weight, W_gcn, b_gcn, W1, b1, W2, b2, W3, b3):
  pad = E_PAD - E
  src = jnp.concatenate([edge_index[0], jnp.zeros((pad,), jnp.int32)])
  dst = jnp.concatenate([edge_index[1], jnp.zeros((pad,), jnp.int32)])
  ew = jnp.concatenate([edges_weight, jnp.zeros((pad,), jnp.float32)])
  zero128 = jnp.zeros((B, HP), jnp.float32)

  ews128 = jnp.broadcast_to(ew[:, None], (E_PAD, HP))
  degp = _scatter_kernel(ews128, dst, zero128)
  dega = degp[:N, 0:1]
  degb = degp[NPAD:NPAD + N, 0:1]

  ht, htp, dinv = _pre_call(x, W_gcn, dega, degb)

  msgs = _gather_kernel(htp, src)
  msgs = _scale_call(msgs, ew[:, None])
  aggp = _scatter_kernel(msgs, dst, zero128)

  return _post_call(aggp[:N], aggp[NPAD:NPAD + N], ht, dinv,
                    b_gcn.reshape(1, H1), W1, b1.reshape(1, H2),
                    W2, b2.reshape(1, H3), W3, b3.reshape(1, C))


# speedup vs baseline: 7.9950x; 1.5893x over previous
import functools

import jax
import jax.numpy as jnp
from jax import lax
from jax.experimental import pallas as pl
from jax.experimental.pallas import tpu as pltpu
from jax.experimental.pallas import tpu_sc as plsc

N = 10000
NPAD = 10240
E = 320000
F_IN = 128
H1, H2, H3, C = 64, 32, 16, 4

HP = 128

NC, NS, L = 2, 16, 16
NW = NC * NS
B = 128
WIN = 80
EPT = B * WIN
E_PAD = EPT * NW
SEG = NPAD // NS

_mesh = plsc.VectorSubcoreMesh(core_axis_name="c", subcore_axis_name="s")


@functools.partial(
    pl.kernel,
    out_type=jax.ShapeDtypeStruct((E_PAD, HP), jnp.float32),
    mesh=_mesh,
    scratch_types=[
        pltpu.VMEM((B,), jnp.int32),
        pltpu.VMEM((B, HP), jnp.float32),
        pltpu.VMEM_SHARED((NPAD, HP), jnp.float32),
        pltpu.SemaphoreType.DMA,
    ],
)
def _gather_kernel(ht_h, src_h, out, idx_v, rows_v, tab_sh, sem):
  c = lax.axis_index("c")
  s = lax.axis_index("s")
  wid = s * NC + c
  base = wid * EPT

  for k in range(SEG // B):
    r0 = s * SEG + k * B
    pltpu.sync_copy(ht_h.at[pl.ds(r0, B)], rows_v)
    pltpu.sync_copy(rows_v, tab_sh.at[pl.ds(r0, B)])
  plsc.subcore_barrier()

  def body(w, carry):
    off = base + w * B
    pltpu.sync_copy(src_h.at[pl.ds(off, B)], idx_v)
    pltpu.async_copy(tab_sh.at[idx_v], rows_v, sem).wait()
    pltpu.sync_copy(rows_v, out.at[pl.ds(off, B)])
    return carry

  lax.fori_loop(0, WIN, body, 0)


@functools.partial(
    pl.kernel,
    out_type=jax.ShapeDtypeStruct((NC * NPAD, HP), jnp.float32),
    mesh=_mesh,
    scratch_types=[
        pltpu.VMEM((B,), jnp.int32),
        pltpu.VMEM((B, HP), jnp.float32),
        pltpu.VMEM_SHARED((NPAD, HP), jnp.float32),
    ],
)
def _scatter_kernel(msg_h, dst_h, zero_h, out, idx_v, val_v, agg_sh):
  c = lax.axis_index("c")
  s = lax.axis_index("s")
  wid = s * NC + c
  base = wid * EPT

  pltpu.sync_copy(zero_h.at[pl.ds(0, B)], val_v)
  for k in range(SEG // B):
    pltpu.sync_copy(val_v, agg_sh.at[pl.ds(s * SEG + k * B, B)])
  plsc.subcore_barrier()

  def body(w, carry):
    off = base + w * B
    pltpu.sync_copy(dst_h.at[pl.ds(off, B)], idx_v)
    pltpu.sync_copy(msg_h.at[pl.ds(off, B)], val_v)
    pltpu.sync_copy(val_v, agg_sh.at[idx_v], add=True)
    return carry

  lax.fori_loop(0, WIN, body, 0)
  plsc.subcore_barrier()
  for k in range(SEG // B):
    pltpu.sync_copy(agg_sh.at[pl.ds(s * SEG + k * B, B)], val_v)
    pltpu.sync_copy(val_v, out.at[pl.ds(c * NPAD + s * SEG + k * B, B)])


def _pre_body(x_ref, w_ref, dega_ref, degb_ref, ht_ref, htp_ref, dinv_ref):
  deg = dega_ref[...] + degb_ref[...] + 1.0
  dinv = lax.rsqrt(deg)
  h = jnp.dot(x_ref[...], w_ref[...], preferred_element_type=jnp.float32)
  hd = h * dinv
  ht_ref[...] = hd
  htp_ref[...] = jnp.concatenate([hd, jnp.zeros_like(hd)], axis=1)
  dinv_ref[...] = dinv


_pre_call = pl.pallas_call(
    _pre_body,
    out_shape=[
        jax.ShapeDtypeStruct((N, H1), jnp.float32),
        jax.ShapeDtypeStruct((N, HP), jnp.float32),
        jax.ShapeDtypeStruct((N, 1), jnp.float32),
    ],
)

_SCALE_ROWS = 4096


def _scale_body(msg_ref, ew_ref, out_ref):
  out_ref[...] = msg_ref[...] * ew_ref[...]


_scale_call = pl.pallas_call(
    _scale_body,
    grid=(E_PAD // _SCALE_ROWS,),
    in_specs=[
        pl.BlockSpec((_SCALE_ROWS, HP), lambda i: (i, 0)),
        pl.BlockSpec((_SCALE_ROWS, 1), lambda i: (i, 0)),
    ],
    out_specs=pl.BlockSpec((_SCALE_ROWS, HP), lambda i: (i, 0)),
    out_shape=jax.ShapeDtypeStruct((E_PAD, HP), jnp.float32),
)


def _post_body(agga_ref, aggb_ref, ht_ref, dinv_ref, bg_ref,
               w1_ref, b1_ref, w2_ref, b2_ref, w3_ref, b3_ref, out_ref):
  a = (agga_ref[:, :H1] + aggb_ref[:, :H1] + ht_ref[...]) * dinv_ref[...]
  a = jnp.maximum(a + bg_ref[...], 0.0)
  a = jnp.maximum(
      jnp.dot(a, w1_ref[...], preferred_element_type=jnp.float32)
      + b1_ref[...], 0.0)
  a = jnp.maximum(
      jnp.dot(a, w2_ref[...], preferred_element_type=jnp.float32)
      + b2_ref[...], 0.0)
  z = jnp.dot(a, w3_ref[...], preferred_element_type=jnp.float32) + b3_ref[...]
  m = jnp.max(z, axis=1, keepdims=True)
  lse = jnp.log(jnp.sum(jnp.exp(z - m), axis=1, keepdims=True)) + m
  out_ref[...] = z - lse


_post_call = pl.pallas_call(
    _post_body,
    out_shape=jax.ShapeDtypeStruct((N, C), jnp.float32),
)


def kernel(x, edge_index, edges_weight, W_gcn, b_gcn, W1, b1, W2, b2, W3, b3):
  pad = E_PAD - E
  src = jnp.concatenate([edge_index[0], jnp.zeros((pad,), jnp.int32)])
  dst = jnp.concatenate([edge_index[1], jnp.zeros((pad,), jnp.int32)])
  ew = jnp.concatenate([edges_weight, jnp.zeros((pad,), jnp.float32)])
  zero128 = jnp.zeros((B, HP), jnp.float32)

  ews128 = jnp.broadcast_to(ew[:, None], (E_PAD, HP))
  degp = _scatter_kernel(ews128, dst, zero128)
  dega = degp[:N, 0:1]
  degb = degp[NPAD:NPAD + N, 0:1]

  ht, htp, dinv = _pre_call(x, W_gcn, dega, degb)

  htp = jnp.pad(htp, ((0, NPAD - N), (0, 0)))
  msgs = _gather_kernel(htp, src)
  msgs = _scale_call(msgs, ew[:, None])
  aggp = _scatter_kernel(msgs, dst, zero128)

  return _post_call(aggp[:N], aggp[NPAD:NPAD + N], ht, dinv,
                    b_gcn.reshape(1, H1), W1, b1.reshape(1, H2),
                    W2, b2.reshape(1, H3), W3, b3.reshape(1, C))
